# Initial kernel scaffold; baseline (speedup 1.0000x reference)
import jax
import jax.numpy as jnp
from jax.experimental import pallas as pl


def kernel(x, W):
    raise NotImplementedError("write your pallas kernel here")


# speedup vs baseline: 2.2561x; 2.2561x over previous
import math

import jax
import jax.numpy as jnp
from jax.experimental import pallas as pl
from jax.experimental.pallas import tpu as pltpu

_CAPACITY = 0.5


def _router_body(B, NT, Tt, T, k,
                 x_ref, w_ref, scores_ref, mask_ref, aux_ref,
                 keys_ref, sums_ref):
    b = pl.program_id(0)
    i = pl.program_id(1)

    xb = x_ref[0]
    w = w_ref[0]
    logits = jnp.dot(xb, w[:, None],
                     preferred_element_type=jnp.float32)[:, 0]
    s = 1.0 / (1.0 + jnp.exp(-logits))
    scores_ref[0, 0, :] = s

    bits = jax.lax.bitcast_convert_type(logits, jnp.int32)
    mag = bits & jnp.int32(0x7FFFFFFF)
    keys_ref[b, i, :] = jnp.where(bits >= 0, bits, -mag)

    @pl.when(i == 0)
    def _():
        sums_ref[b] = 0.0

    sums_ref[b] = sums_ref[b] + jnp.sum(s)

    @pl.when((b == B - 1) & (i == NT - 1))
    def _():
        keys = keys_ref[:, :, :]
        K = jnp.int32(k)

        def cnt_ge(c):
            return jnp.sum((keys >= c).astype(jnp.int32), axis=(1, 2),
                           keepdims=True)

        zero = jnp.zeros((B, 1, 1), jnp.int32)
        int_min = jnp.full((B, 1, 1), -2147483648, jnp.int32)
        acc0 = jnp.where(cnt_ge(zero) >= K, zero, int_min)

        def tbody(t, acc):
            cand = acc | (jnp.int32(1) << (jnp.int32(30) - t))
            return jnp.where(cnt_ge(cand) >= K, cand, acc)

        thresh = jax.lax.fori_loop(0, 31, tbody, acc0)

        n_gt = jnp.sum((keys > thresh).astype(jnp.int32), axis=(1, 2),
                       keepdims=True)
        need = K - n_gt
        eq = keys == thresh
        pos = (jax.lax.broadcasted_iota(jnp.int32, (B, NT, Tt), 1) * Tt
               + jax.lax.broadcasted_iota(jnp.int32, (B, NT, Tt), 2))

        def cbody(t, acc):
            cand = acc | (jnp.int32(1) << (jnp.int32(12) - t))
            f = jnp.sum((eq & (pos < cand)).astype(jnp.int32), axis=(1, 2),
                        keepdims=True)
            return jnp.where(f < need, cand, acc)

        cmax = jax.lax.fori_loop(0, 13, cbody, jnp.zeros((B, 1, 1), jnp.int32))
        cutoff = cmax + 1
        sel = (keys > thresh) | (eq & (pos < cutoff))
        mask_ref[:, :, :] = sel.astype(jnp.float32)

        if B > 1:
            means = [sums_ref[r] / T for r in range(B)]
            mbar = sum(means) / B
            var = sum((m - mbar) ** 2 for m in means) / (B - 1)
        else:
            var = jnp.float32(0.0)
        aux_ref[:, :] = jnp.reshape(var, (1, 1))


def kernel(x, W):
    B, T, D = x.shape
    Tt = 1024
    NT = T // Tt
    k = max(1, min(T, math.ceil(_CAPACITY * T)))

    def body(x_ref, w_ref, scores_ref, mask_ref, aux_ref, keys_ref, sums_ref):
        _router_body(B, NT, Tt, T, k, x_ref, w_ref, scores_ref, mask_ref,
                     aux_ref, keys_ref, sums_ref)

    scores3, maskf, aux = pl.pallas_call(
        body,
        grid=(B, NT),
        in_specs=[
            pl.BlockSpec((1, Tt, D), lambda b, i: (b, i, 0)),
            pl.BlockSpec((1, D), lambda b, i: (0, 0)),
        ],
        out_specs=[
            pl.BlockSpec((1, 1, Tt), lambda b, i: (b * NT + i, 0, 0)),
            pl.BlockSpec((B, NT, Tt), lambda b, i: (0, 0, 0)),
            pl.BlockSpec((1, 1), lambda b, i: (0, 0)),
        ],
        out_shape=[
            jax.ShapeDtypeStruct((B * NT, 1, Tt), jnp.float32),
            jax.ShapeDtypeStruct((B, NT, Tt), jnp.float32),
            jax.ShapeDtypeStruct((1, 1), jnp.float32),
        ],
        scratch_shapes=[
            pltpu.VMEM((B, NT, Tt), jnp.int32),
            pltpu.SMEM((B,), jnp.float32),
        ],
    )(x, W)

    router_weights = scores3.reshape(B, T)[..., None]
    selected_mask = maskf.reshape(B, T).astype(jnp.bool_)
    aux_loss = aux[0, 0]
    return (router_weights, selected_mask, aux_loss)


# speedup vs baseline: 2.6692x; 1.1831x over previous
import math

import jax
import jax.numpy as jnp
from jax.experimental import pallas as pl
from jax.experimental.pallas import tpu as pltpu

_CAPACITY = 0.5
_LANES = 128


def _router_body(B, NT, Tt, T, k,
                 x_ref, wrep_ref, scores_ref, mask_ref, aux_ref,
                 keys_ref, sums_ref):
    b = pl.program_id(0)
    i = pl.program_id(1)
    SL = Tt // _LANES

    xb = x_ref[0]
    rep = jnp.dot(xb, wrep_ref[:, :],
                  preferred_element_type=jnp.float32)
    rep3 = rep.reshape(SL, _LANES, _LANES)
    r1 = jax.lax.broadcasted_iota(jnp.int32, (SL, _LANES, _LANES), 1)
    c2 = jax.lax.broadcasted_iota(jnp.int32, (SL, _LANES, _LANES), 2)
    logits = jnp.sum(jnp.where(r1 == c2, rep3, 0.0), axis=1)

    s = 1.0 / (1.0 + jnp.exp(-logits))
    scores_ref[0] = s

    bits = jax.lax.bitcast_convert_type(logits, jnp.int32)
    mag = bits & jnp.int32(0x7FFFFFFF)
    keys_ref[b, i] = jnp.where(bits >= 0, bits, -mag)

    @pl.when(i == 0)
    def _():
        sums_ref[b] = 0.0

    sums_ref[b] = sums_ref[b] + jnp.sum(s)

    @pl.when((b == B - 1) & (i == NT - 1))
    def _():
        keys = keys_ref[:, :, :, :]
        K = jnp.int32(k)

        def cnt_ge(c):
            return jnp.sum((keys >= c).astype(jnp.int32), axis=(1, 2, 3),
                           keepdims=True)

        zero = jnp.zeros((B, 1, 1, 1), jnp.int32)
        int_min = jnp.full((B, 1, 1, 1), -2147483648, jnp.int32)
        acc0 = jnp.where(cnt_ge(zero) >= K, zero, int_min)

        def tbody(t, acc):
            cand = acc | (jnp.int32(1) << (jnp.int32(30) - t))
            return jnp.where(cnt_ge(cand) >= K, cand, acc)

        thresh = jax.lax.fori_loop(0, 31, tbody, acc0)

        n_gt = jnp.sum((keys > thresh).astype(jnp.int32), axis=(1, 2, 3),
                       keepdims=True)
        need = K - n_gt
        eq = keys == thresh
        sh = (B, NT, SL, _LANES)
        pos = (jax.lax.broadcasted_iota(jnp.int32, sh, 1) * Tt
               + jax.lax.broadcasted_iota(jnp.int32, sh, 2) * _LANES
               + jax.lax.broadcasted_iota(jnp.int32, sh, 3))

        def cbody(t, acc):
            cand = acc | (jnp.int32(1) << (jnp.int32(12) - t))
            f = jnp.sum((eq & (pos < cand)).astype(jnp.int32), axis=(1, 2, 3),
                        keepdims=True)
            return jnp.where(f < need, cand, acc)

        cmax = jax.lax.fori_loop(0, 13, cbody,
                                 jnp.zeros((B, 1, 1, 1), jnp.int32))
        cutoff = cmax + 1
        sel = (keys > thresh) | (eq & (pos < cutoff))
        mask_ref[:, :, :, :] = sel.astype(jnp.float32)

        if B > 1:
            means = [sums_ref[r] / T for r in range(B)]
            mbar = sum(means) / B
            var = sum((m - mbar) ** 2 for m in means) / (B - 1)
        else:
            var = jnp.float32(0.0)
        aux_ref[:, :] = jnp.reshape(var, (1, 1))


def kernel(x, W):
    B, T, D = x.shape
    Tt = 1024
    NT = T // Tt
    SL = Tt // _LANES
    k = max(1, min(T, math.ceil(_CAPACITY * T)))

    wrep = jnp.broadcast_to(W.reshape(D, 1), (D, _LANES))

    def body(x_ref, wrep_ref, scores_ref, mask_ref, aux_ref, keys_ref,
             sums_ref):
        _router_body(B, NT, Tt, T, k, x_ref, wrep_ref, scores_ref, mask_ref,
                     aux_ref, keys_ref, sums_ref)

    scores4, maskf, aux = pl.pallas_call(
        body,
        grid=(B, NT),
        in_specs=[
            pl.BlockSpec((1, Tt, D), lambda b, i: (b, i, 0)),
            pl.BlockSpec((D, _LANES), lambda b, i: (0, 0)),
        ],
        out_specs=[
            pl.BlockSpec((1, SL, _LANES), lambda b, i: (b * NT + i, 0, 0)),
            pl.BlockSpec((B, NT, SL, _LANES), lambda b, i: (0, 0, 0, 0)),
            pl.BlockSpec((1, 1), lambda b, i: (0, 0)),
        ],
        out_shape=[
            jax.ShapeDtypeStruct((B * NT, SL, _LANES), jnp.float32),
            jax.ShapeDtypeStruct((B, NT, SL, _LANES), jnp.float32),
            jax.ShapeDtypeStruct((1, 1), jnp.float32),
        ],
        scratch_shapes=[
            pltpu.VMEM((B, NT, SL, _LANES), jnp.int32),
            pltpu.SMEM((B,), jnp.float32),
        ],
    )(x, wrep)

    router_weights = scores4.reshape(B, T)[..., None]
    selected_mask = maskf.reshape(B, T).astype(jnp.bool_)
    aux_loss = aux[0, 0]
    return (router_weights, selected_mask, aux_loss)


# speedup vs baseline: 3.2483x; 1.2170x over previous
import math

import jax
import jax.numpy as jnp
from jax.experimental import pallas as pl
from jax.experimental.pallas import tpu as pltpu

_CAPACITY = 0.5
_LANES = 128


def _router_body(B, NT, Tt, T, k,
                 x_ref, wrep_ref, scores_ref, mask_ref, aux_ref,
                 keys_ref, sums_ref):
    b = pl.program_id(0)
    i = pl.program_id(1)
    SL = Tt // _LANES

    xb = x_ref[0]
    rep = jnp.dot(xb, wrep_ref[:, :],
                  preferred_element_type=jnp.float32)
    rep3 = rep.reshape(SL, _LANES, _LANES)
    r1 = jax.lax.broadcasted_iota(jnp.int32, (SL, _LANES, _LANES), 1)
    c2 = jax.lax.broadcasted_iota(jnp.int32, (SL, _LANES, _LANES), 2)
    logits = jnp.sum(jnp.where(r1 == c2, rep3, 0.0), axis=1)

    s = 1.0 / (1.0 + jnp.exp(-logits))
    scores_ref[0] = s

    bits = jax.lax.bitcast_convert_type(logits, jnp.int32)
    mag = bits & jnp.int32(0x7FFFFFFF)
    keys_ref[b, i] = jnp.where(bits >= 0, bits, -mag)

    @pl.when(i == 0)
    def _():
        sums_ref[b] = 0.0

    sums_ref[b] = sums_ref[b] + jnp.sum(s)

    @pl.when((b == B - 1) & (i == NT - 1))
    def _():
        keys = keys_ref[:, :, :, :]
        K = jnp.int32(k)

        def cnt_ge(c):
            return jnp.sum((keys >= c).astype(jnp.int32), axis=(1, 2, 3),
                           keepdims=True)

        zero = jnp.zeros((B, 1, 1, 1), jnp.int32)
        int_min = jnp.full((B, 1, 1, 1), -2147483648, jnp.int32)
        acc0 = jnp.where(cnt_ge(zero) >= K, zero, int_min)
        cand30 = acc0 | (jnp.int32(1) << 30)
        acc0 = jnp.where(cnt_ge(cand30) >= K, cand30, acc0)

        def tbody(t, acc):
            s = jnp.int32(28) - 2 * t
            c1 = acc | (jnp.int32(1) << s)
            c2 = acc | (jnp.int32(2) << s)
            c3 = acc | (jnp.int32(3) << s)
            acc = jnp.where(cnt_ge(c1) >= K, c1, acc)
            acc = jnp.where(cnt_ge(c2) >= K, c2, acc)
            return jnp.where(cnt_ge(c3) >= K, c3, acc)

        thresh = jax.lax.fori_loop(0, 15, tbody, acc0)

        n_gt = jnp.sum((keys > thresh).astype(jnp.int32), axis=(1, 2, 3),
                       keepdims=True)
        need = K - n_gt
        eq = keys == thresh
        sh = (B, NT, SL, _LANES)
        pos = (jax.lax.broadcasted_iota(jnp.int32, sh, 1) * Tt
               + jax.lax.broadcasted_iota(jnp.int32, sh, 2) * _LANES
               + jax.lax.broadcasted_iota(jnp.int32, sh, 3))

        def f_lt(cand):
            return jnp.sum((eq & (pos < cand)).astype(jnp.int32), axis=(1, 2, 3),
                           keepdims=True)

        czero = jnp.zeros((B, 1, 1, 1), jnp.int32)
        cand12 = czero | (jnp.int32(1) << 12)
        cinit = jnp.where(f_lt(cand12) < need, cand12, czero)

        def cbody(t, acc):
            s = jnp.int32(10) - 2 * t
            c1 = acc | (jnp.int32(1) << s)
            c2 = acc | (jnp.int32(2) << s)
            c3 = acc | (jnp.int32(3) << s)
            acc = jnp.where(f_lt(c1) < need, c1, acc)
            acc = jnp.where(f_lt(c2) < need, c2, acc)
            return jnp.where(f_lt(c3) < need, c3, acc)

        cmax = jax.lax.fori_loop(0, 6, cbody, cinit)
        cutoff = cmax + 1
        sel = (keys > thresh) | (eq & (pos < cutoff))
        mask_ref[:, :, :, :] = sel.astype(jnp.float32)

        if B > 1:
            means = [sums_ref[r] / T for r in range(B)]
            mbar = sum(means) / B
            var = sum((m - mbar) ** 2 for m in means) / (B - 1)
        else:
            var = jnp.float32(0.0)
        aux_ref[:, :] = jnp.reshape(var, (1, 1))


def kernel(x, W):
    B, T, D = x.shape
    Tt = 2048
    NT = T // Tt
    SL = Tt // _LANES
    k = max(1, min(T, math.ceil(_CAPACITY * T)))

    wrep = jnp.broadcast_to(W.reshape(D, 1), (D, _LANES))

    def body(x_ref, wrep_ref, scores_ref, mask_ref, aux_ref, keys_ref,
             sums_ref):
        _router_body(B, NT, Tt, T, k, x_ref, wrep_ref, scores_ref, mask_ref,
                     aux_ref, keys_ref, sums_ref)

    scores4, maskf, aux = pl.pallas_call(
        body,
        grid=(B, NT),
        in_specs=[
            pl.BlockSpec((1, Tt, D), lambda b, i: (b, i, 0)),
            pl.BlockSpec((D, _LANES), lambda b, i: (0, 0)),
        ],
        out_specs=[
            pl.BlockSpec((1, SL, _LANES), lambda b, i: (b * NT + i, 0, 0)),
            pl.BlockSpec((B, NT, SL, _LANES), lambda b, i: (0, 0, 0, 0)),
            pl.BlockSpec((1, 1), lambda b, i: (0, 0)),
        ],
        out_shape=[
            jax.ShapeDtypeStruct((B * NT, SL, _LANES), jnp.float32),
            jax.ShapeDtypeStruct((B, NT, SL, _LANES), jnp.float32),
            jax.ShapeDtypeStruct((1, 1), jnp.float32),
        ],
        scratch_shapes=[
            pltpu.VMEM((B, NT, SL, _LANES), jnp.int32),
            pltpu.SMEM((B,), jnp.float32),
        ],
    )(x, wrep)

    router_weights = scores4.reshape(B, T)[..., None]
    selected_mask = maskf.reshape(B, T).astype(jnp.bool_)
    aux_loss = aux[0, 0]
    return (router_weights, selected_mask, aux_loss)
